# baseline (speedup 1.0000x reference)
import functools

import jax
import jax.numpy as jnp
from jax import lax
from jax.experimental import pallas as pl
from jax.experimental.pallas import tpu as pltpu, tpu_sc as plsc

N = 10000
E = 320000
D = 128
ED = 32
NC = 2
NS = 16
NW = NC * NS
PER_W = E // NW
C = 32
G = C // 16
FCH = PER_W // C
PAIRS = FCH // 2
TAIL_E = PER_W - FCH * C
ROWS_PER_TILE = 624
TAIL_ROW = NS * ROWS_PER_TILE
ISQD = 1.0 / float(D) ** 0.5


def _sc_edge_pass(i0, i1, q, k, v, eigs, lam):
    mesh = plsc.VectorSubcoreMesh(core_axis_name="c", subcore_axis_name="s")

    buf_set = [
        pltpu.VMEM((C,), jnp.int32),
        pltpu.VMEM((C,), jnp.int32),
        pltpu.VMEM((C,), jnp.int32),
        pltpu.VMEM((C,), jnp.int32),
        pltpu.VMEM((C, D), jnp.float32),
        pltpu.VMEM((C, D), jnp.float32),
        pltpu.VMEM((C, D), jnp.float32),
        pltpu.VMEM((C, D), jnp.float32),
        pltpu.VMEM((C, D), jnp.float32),
        pltpu.VMEM((C,), jnp.float32),
    ]

    @functools.partial(
        pl.kernel,
        mesh=mesh,
        out_type=[
            jax.ShapeDtypeStruct((NC * N, D), jnp.float32),
            jax.ShapeDtypeStruct((NC * N,), jnp.float32),
        ],
        scratch_types=buf_set + buf_set + [
            pltpu.VMEM((TAIL_E,), jnp.int32),
            pltpu.VMEM((TAIL_E,), jnp.int32),
            pltpu.VMEM((16,), jnp.float32),
            pltpu.VMEM((16,), jnp.float32),
            pltpu.VMEM_SHARED((N, D), jnp.float32),
            pltpu.VMEM_SHARED((N,), jnp.float32),
            pltpu.SemaphoreType.DMA,
            pltpu.SemaphoreType.DMA,
            pltpu.SemaphoreType.DMA,
            pltpu.SemaphoreType.DMA,
            pltpu.SemaphoreType.DMA,
            pltpu.SemaphoreType.DMA,
        ],
    )
    def body(i0_hbm, i1_hbm, q_hbm, k_hbm, v_hbm, eigs_hbm, lam_hbm,
             acc_out, den_out, *refs):
        seta = refs[0:10]
        setb = refs[10:20]
        (i0_t, i1_t, z16buf, lam_v, acc_sh, den_sh,
         gsem_a, gsem_b, ssem_a, ssem_b, isem_a, isem_b) = refs[20:]
        gsems = (gsem_a, gsem_b)
        ssems = (ssem_a, ssem_b)
        isems = (isem_a, isem_b)
        sets = (seta, setb)

        cid = lax.axis_index("c")
        sid = lax.axis_index("s")
        wid = sid * NC + cid
        ebase = wid * PER_W
        zeros16 = jnp.zeros((16,), jnp.float32)

        q_a = seta[4]

        def zero_qa(r, carry):
            for j in range(D // 16):
                q_a[r, pl.ds(16 * j, 16)] = zeros16
            return carry

        lax.fori_loop(0, 16, zero_qa, 0)
        z16buf[...] = zeros16

        def zero_acc(t, carry):
            pltpu.sync_copy(q_a.at[pl.ds(0, 16)],
                            acc_sh.at[pl.ds(sid * ROWS_PER_TILE + t * 16, 16)])
            pltpu.sync_copy(z16buf,
                            den_sh.at[pl.ds(sid * ROWS_PER_TILE + t * 16, 16)])
            return carry

        lax.fori_loop(0, ROWS_PER_TILE // 16, zero_acc, 0)

        @pl.when(sid == NS - 1)
        def _():
            pltpu.sync_copy(q_a.at[pl.ds(0, 16)], acc_sh.at[pl.ds(TAIL_ROW, 16)])
            pltpu.sync_copy(z16buf, den_sh.at[pl.ds(TAIL_ROW, 16)])

        pltpu.sync_copy(lam_hbm, lam_v)
        plsc.subcore_barrier()

        lam_e = jnp.exp(lam_v[...])
        lanes = lax.iota(jnp.int32, 16)
        perms = [jnp.bitwise_xor(lanes, sh)[:, None] for sh in (1, 2, 4, 8)]
        _dnums = lax.GatherDimensionNumbers(
            offset_dims=(), collapsed_slice_dims=(0,), start_index_map=(0,))

        def _lanesum(x):
            for p in perms:
                x = x + lax.gather(
                    x, p, dimension_numbers=_dnums, slice_sizes=(1,),
                    mode=lax.GatherScatterMode.PROMISE_IN_BOUNDS)
            return x

        def make_score(si):
            q_rows, k_rows, e0_rows, e1_rows = sets[si][4:8]
            w_rows, vals_buf = sets[si][8], sets[si][9]

            def score_body(g, carry):
                merged = zeros16
                for t in range(16):
                    c = g * 16 + t
                    accv = zeros16
                    for j in range(D // 16):
                        accv = accv + q_rows[c, pl.ds(16 * j, 16)] * k_rows[c, pl.ds(16 * j, 16)]
                    acce = zeros16
                    for j in range(ED // 16):
                        acce = acce + e0_rows[c, pl.ds(16 * j, 16)] * e1_rows[c, pl.ds(16 * j, 16)]
                    s = _lanesum(accv * ISQD + acce * lam_e)
                    vals = jnp.minimum(jnp.exp(s), 5.0)
                    for j in range(D // 16):
                        w_rows[c, pl.ds(16 * j, 16)] = w_rows[c, pl.ds(16 * j, 16)] * vals
                    merged = jnp.where(lanes == t, vals, merged)
                vals_buf[pl.ds(g * 16, 16)] = merged
                return carry

            return score_body

        scores = [make_score(0), make_score(1)]

        def issue_idx_load(si, ch):
            s = sets[si]
            base = ebase + ch * C
            pltpu.async_copy(i0_hbm.at[pl.ds(base, C)], s[0], isems[si])
            pltpu.async_copy(i1_hbm.at[pl.ds(base, C)], s[1], isems[si])

        def wait_idx_load(si):
            s = sets[si]
            pltpu.make_async_copy(i0_hbm.at[pl.ds(0, C)], s[0], isems[si]).wait()
            pltpu.make_async_copy(i1_hbm.at[pl.ds(0, C)], s[1], isems[si]).wait()

        def issue_gathers(si):
            s = sets[si]
            gsem = gsems[si]
            pltpu.async_copy(q_hbm.at[s[0]], s[4], gsem)
            pltpu.async_copy(k_hbm.at[s[1]], s[5], gsem)
            pltpu.async_copy(eigs_hbm.at[s[0]], s[6], gsem)
            pltpu.async_copy(eigs_hbm.at[s[1]], s[7], gsem)

        def wait_gathers(si):
            s = sets[si]
            gsem = gsems[si]
            pltpu.make_async_copy(q_hbm.at[s[0]], s[4], gsem).wait()
            pltpu.make_async_copy(k_hbm.at[s[1]], s[5], gsem).wait()
            pltpu.make_async_copy(eigs_hbm.at[s[0]], s[6], gsem).wait()
            pltpu.make_async_copy(eigs_hbm.at[s[1]], s[7], gsem).wait()

        def issue_v_gather(si):
            s = sets[si]
            pltpu.async_copy(v_hbm.at[s[3]], s[8], gsems[si])

        def wait_v_gather(si):
            s = sets[si]
            pltpu.make_async_copy(v_hbm.at[s[3]], s[8], gsems[si]).wait()

        def issue_scatters(si):
            s = sets[si]
            pltpu.async_copy(s[8], acc_sh.at[s[2]], add=True, sem=ssems[si])
            pltpu.async_copy(s[9], den_sh.at[s[2]], add=True, sem=ssems[si])

        def wait_scatters(si):
            s = sets[si]
            pltpu.make_async_copy(s[8], acc_sh.at[s[2]], ssems[si]).wait()
            pltpu.make_async_copy(s[9], den_sh.at[s[2]], ssems[si]).wait()

        def copy_idx(si):
            s = sets[si]
            for j in range(C // 16):
                s[2][pl.ds(16 * j, 16)] = s[0][pl.ds(16 * j, 16)]
                s[3][pl.ds(16 * j, 16)] = s[1][pl.ds(16 * j, 16)]

        def process(si, i):
            wait_gathers(si)

            @pl.when(i > 0)
            def _():
                wait_scatters(si)

            copy_idx(si)
            issue_v_gather(si)

            @pl.when(i < PAIRS - 1)
            def _():
                issue_idx_load(si, 2 * i + si + 2)

            wait_v_gather(si)
            lax.fori_loop(0, G, scores[si], 0)
            issue_scatters(si)

            @pl.when(i < PAIRS - 1)
            def _():
                wait_idx_load(si)
                issue_gathers(si)

        for si in (0, 1):
            s = sets[si]
            base = ebase + si * C
            pltpu.sync_copy(i0_hbm.at[pl.ds(base, C)], s[0])
            pltpu.sync_copy(i1_hbm.at[pl.ds(base, C)], s[1])
            issue_gathers(si)

        def pair_body(i, carry):
            process(0, i)
            process(1, i)
            return carry

        lax.fori_loop(0, PAIRS, pair_body, 0)

        wait_scatters(0)
        wait_scatters(1)

        q_aT, k_aT, e0_aT, e1_aT, w_aT, vals_aT = seta[4:10]
        base = ebase + FCH * C
        pltpu.sync_copy(i0_hbm.at[pl.ds(base, TAIL_E)], i0_t)
        pltpu.sync_copy(i1_hbm.at[pl.ds(base, TAIL_E)], i1_t)
        pltpu.async_copy(q_hbm.at[i0_t], q_aT.at[pl.ds(0, TAIL_E)], gsem_a).wait()
        pltpu.async_copy(k_hbm.at[i1_t], k_aT.at[pl.ds(0, TAIL_E)], gsem_a).wait()
        pltpu.async_copy(v_hbm.at[i1_t], w_aT.at[pl.ds(0, TAIL_E)], gsem_a).wait()
        pltpu.async_copy(eigs_hbm.at[i0_t], e0_aT.at[pl.ds(0, TAIL_E)], gsem_a).wait()
        pltpu.async_copy(eigs_hbm.at[i1_t], e1_aT.at[pl.ds(0, TAIL_E)], gsem_a).wait()
        scores[0](0, 0)
        pltpu.async_copy(w_aT.at[pl.ds(0, TAIL_E)], acc_sh.at[i0_t], add=True, sem=ssem_a).wait()
        pltpu.async_copy(vals_aT.at[pl.ds(0, TAIL_E)], den_sh.at[i0_t], add=True, sem=ssem_a).wait()

        plsc.subcore_barrier()
        pltpu.sync_copy(
            acc_sh.at[pl.ds(sid * ROWS_PER_TILE, ROWS_PER_TILE)],
            acc_out.at[pl.ds(cid * N + sid * ROWS_PER_TILE, ROWS_PER_TILE)],
        )

        def den_out_copy(t, carry):
            r = sid * ROWS_PER_TILE + t * 16
            pltpu.sync_copy(den_sh.at[pl.ds(r, 16)], z16buf)
            pltpu.sync_copy(z16buf, den_out.at[pl.ds(cid * N + r, 16)])
            return carry

        lax.fori_loop(0, ROWS_PER_TILE // 16, den_out_copy, 0)

        @pl.when(sid == NS - 1)
        def _():
            pltpu.sync_copy(
                acc_sh.at[pl.ds(TAIL_ROW, N - TAIL_ROW)],
                acc_out.at[pl.ds(cid * N + TAIL_ROW, N - TAIL_ROW)],
            )
            pltpu.sync_copy(den_sh.at[pl.ds(TAIL_ROW, 16)], z16buf)
            pltpu.sync_copy(z16buf, den_out.at[pl.ds(cid * N + TAIL_ROW, 16)])

    return body(i0, i1, q, k, v, eigs, lam)


def _tc_combine(acc, den2):
    blk = 2000
    nb = N // blk

    def body(a0, a1, dn, o):
        d = jnp.sum(dn[...], axis=1)
        d = jnp.where(d == 0.0, 1.0, d)
        o[...] = (a0[...] + a1[...]) / d[:, None]

    return pl.pallas_call(
        body,
        grid=(nb,),
        in_specs=[
            pl.BlockSpec((blk, D), lambda i: (i, 0)),
            pl.BlockSpec((blk, D), lambda i: (i + nb, 0)),
            pl.BlockSpec((blk, NC), lambda i: (i, 0)),
        ],
        out_specs=pl.BlockSpec((blk, D), lambda i: (i, 0)),
        out_shape=jax.ShapeDtypeStruct((N, D), jnp.float32),
    )(acc, acc, den2)


def kernel(q, k, v, indices, eigs, lambda0):
    i0 = indices[0].astype(jnp.int32)
    i1 = indices[1].astype(jnp.int32)
    lam = jnp.full((16,), lambda0[0], jnp.float32)
    eigs_p = jnp.pad(eigs, ((0, 0), (0, D - ED)))
    acc, den = _sc_edge_pass(i0, i1, q, k, v, eigs_p, lam)
    return _tc_combine(acc, den.reshape(NC, N).T)


# speedup vs baseline: 1.1165x; 1.1165x over previous
import functools

import jax
import jax.numpy as jnp
from jax import lax
from jax.experimental import pallas as pl
from jax.experimental.pallas import tpu as pltpu, tpu_sc as plsc

N = 10000
E = 320000
D = 128
ED = 32
NC = 2
NS = 16
NW = NC * NS
PER_W = E // NW
C = 32
G = C // 16
FCH = PER_W // C
PAIRS = FCH // 2
TAIL_E = PER_W - FCH * C
ROWS_PER_TILE = 624
TAIL_ROW = NS * ROWS_PER_TILE
ISQD = 1.0 / float(D) ** 0.5


def _sc_edge_pass(i0, i1, q, k, v, eigs, lam):
    mesh = plsc.VectorSubcoreMesh(core_axis_name="c", subcore_axis_name="s")

    buf_set = [
        pltpu.VMEM((C,), jnp.int32),
        pltpu.VMEM((C,), jnp.int32),
        pltpu.VMEM((C,), jnp.int32),
        pltpu.VMEM((C,), jnp.int32),
        pltpu.VMEM((C, D), jnp.float32),
        pltpu.VMEM((C, D), jnp.float32),
        pltpu.VMEM((C, D), jnp.float32),
        pltpu.VMEM((C, D), jnp.float32),
        pltpu.VMEM((C, D), jnp.float32),
        pltpu.VMEM((C,), jnp.float32),
    ]

    @functools.partial(
        pl.kernel,
        mesh=mesh,
        out_type=[
            jax.ShapeDtypeStruct((NC * N, D), jnp.float32),
            jax.ShapeDtypeStruct((NC * N,), jnp.float32),
        ],
        scratch_types=buf_set + buf_set + [
            pltpu.VMEM((TAIL_E,), jnp.int32),
            pltpu.VMEM((TAIL_E,), jnp.int32),
            pltpu.VMEM((16,), jnp.float32),
            pltpu.VMEM((16,), jnp.float32),
            pltpu.VMEM_SHARED((N, D), jnp.float32),
            pltpu.VMEM_SHARED((N,), jnp.float32),
            pltpu.SemaphoreType.DMA,
            pltpu.SemaphoreType.DMA,
            pltpu.SemaphoreType.DMA,
            pltpu.SemaphoreType.DMA,
            pltpu.SemaphoreType.DMA,
            pltpu.SemaphoreType.DMA,
        ],
    )
    def body(i0_hbm, i1_hbm, q_hbm, k_hbm, v_hbm, eigs_hbm, lam_hbm,
             acc_out, den_out, *refs):
        seta = refs[0:10]
        setb = refs[10:20]
        (i0_t, i1_t, z16buf, lam_v, acc_sh, den_sh,
         gsem_a, gsem_b, ssem_a, ssem_b, isem_a, isem_b) = refs[20:]
        gsems = (gsem_a, gsem_b)
        ssems = (ssem_a, ssem_b)
        isems = (isem_a, isem_b)
        sets = (seta, setb)

        cid = lax.axis_index("c")
        sid = lax.axis_index("s")
        wid = sid * NC + cid
        ebase = wid * PER_W
        zeros16 = jnp.zeros((16,), jnp.float32)

        q_a = seta[4]

        def zero_qa(r, carry):
            for j in range(D // 16):
                q_a[r, pl.ds(16 * j, 16)] = zeros16
            return carry

        lax.fori_loop(0, 16, zero_qa, 0)
        z16buf[...] = zeros16

        def zero_acc(t, carry):
            pltpu.sync_copy(q_a.at[pl.ds(0, 16)],
                            acc_sh.at[pl.ds(sid * ROWS_PER_TILE + t * 16, 16)])
            pltpu.sync_copy(z16buf,
                            den_sh.at[pl.ds(sid * ROWS_PER_TILE + t * 16, 16)])
            return carry

        lax.fori_loop(0, ROWS_PER_TILE // 16, zero_acc, 0)

        @pl.when(sid == NS - 1)
        def _():
            pltpu.sync_copy(q_a.at[pl.ds(0, 16)], acc_sh.at[pl.ds(TAIL_ROW, 16)])
            pltpu.sync_copy(z16buf, den_sh.at[pl.ds(TAIL_ROW, 16)])

        pltpu.sync_copy(lam_hbm, lam_v)
        plsc.subcore_barrier()

        lam_e = jnp.exp(lam_v[...])
        lanes = lax.iota(jnp.int32, 16)
        perms = [jnp.bitwise_xor(lanes, sh)[:, None] for sh in (1, 2, 4, 8)]
        _dnums = lax.GatherDimensionNumbers(
            offset_dims=(), collapsed_slice_dims=(0,), start_index_map=(0,))

        def _lanesum(x):
            for p in perms:
                x = x + lax.gather(
                    x, p, dimension_numbers=_dnums, slice_sizes=(1,),
                    mode=lax.GatherScatterMode.PROMISE_IN_BOUNDS)
            return x

        def make_score(si):
            q_rows, k_rows, e0_rows, e1_rows = sets[si][4:8]
            vals_buf = sets[si][9]

            def score_body(g, carry):
                merged = zeros16
                for t in range(16):
                    c = g * 16 + t
                    accv = zeros16
                    for j in range(D // 16):
                        accv = accv + q_rows[c, pl.ds(16 * j, 16)] * k_rows[c, pl.ds(16 * j, 16)]
                    acce = zeros16
                    for j in range(ED // 16):
                        acce = acce + e0_rows[c, pl.ds(16 * j, 16)] * e1_rows[c, pl.ds(16 * j, 16)]
                    s = _lanesum(accv * ISQD + acce * lam_e)
                    vals = jnp.minimum(jnp.exp(s), 5.0)
                    e0_rows[c, pl.ds(ED, 16)] = vals
                    merged = jnp.where(lanes == t, vals, merged)
                vals_buf[pl.ds(g * 16, 16)] = merged
                return carry

            return score_body

        def make_scale(si):
            w_rows, e0_rows = sets[si][8], sets[si][6]

            def scale_body(g, carry):
                for t in range(16):
                    c = g * 16 + t
                    vals = e0_rows[c, pl.ds(ED, 16)]
                    for j in range(D // 16):
                        w_rows[c, pl.ds(16 * j, 16)] = w_rows[c, pl.ds(16 * j, 16)] * vals
                return carry

            return scale_body

        scores = [make_score(0), make_score(1)]
        scales = [make_scale(0), make_scale(1)]

        def issue_idx_load(si, ch):
            s = sets[si]
            base = ebase + ch * C
            pltpu.async_copy(i0_hbm.at[pl.ds(base, C)], s[0], isems[si])
            pltpu.async_copy(i1_hbm.at[pl.ds(base, C)], s[1], isems[si])

        def wait_idx_load(si):
            s = sets[si]
            pltpu.make_async_copy(i0_hbm.at[pl.ds(0, C)], s[0], isems[si]).wait()
            pltpu.make_async_copy(i1_hbm.at[pl.ds(0, C)], s[1], isems[si]).wait()

        def issue_gathers(si):
            s = sets[si]
            gsem = gsems[si]
            pltpu.async_copy(q_hbm.at[s[0]], s[4], gsem)
            pltpu.async_copy(k_hbm.at[s[1]], s[5], gsem)
            pltpu.async_copy(eigs_hbm.at[s[0]], s[6], gsem)
            pltpu.async_copy(eigs_hbm.at[s[1]], s[7], gsem)

        def wait_gathers(si):
            s = sets[si]
            gsem = gsems[si]
            pltpu.make_async_copy(q_hbm.at[s[0]], s[4], gsem).wait()
            pltpu.make_async_copy(k_hbm.at[s[1]], s[5], gsem).wait()
            pltpu.make_async_copy(eigs_hbm.at[s[0]], s[6], gsem).wait()
            pltpu.make_async_copy(eigs_hbm.at[s[1]], s[7], gsem).wait()

        def issue_v_gather(si):
            s = sets[si]
            pltpu.async_copy(v_hbm.at[s[3]], s[8], gsems[si])

        def wait_v_gather(si):
            s = sets[si]
            pltpu.make_async_copy(v_hbm.at[s[3]], s[8], gsems[si]).wait()

        def issue_scatters(si):
            s = sets[si]
            pltpu.async_copy(s[8], acc_sh.at[s[2]], add=True, sem=ssems[si])
            pltpu.async_copy(s[9], den_sh.at[s[2]], add=True, sem=ssems[si])

        def wait_scatters(si):
            s = sets[si]
            pltpu.make_async_copy(s[8], acc_sh.at[s[2]], ssems[si]).wait()
            pltpu.make_async_copy(s[9], den_sh.at[s[2]], ssems[si]).wait()

        def copy_idx(si):
            s = sets[si]
            for j in range(C // 16):
                s[2][pl.ds(16 * j, 16)] = s[0][pl.ds(16 * j, 16)]
                s[3][pl.ds(16 * j, 16)] = s[1][pl.ds(16 * j, 16)]

        def process(si, i):
            wait_gathers(si)

            @pl.when(i > 0)
            def _():
                wait_scatters(si)

            copy_idx(si)
            issue_v_gather(si)

            @pl.when(i < PAIRS - 1)
            def _():
                issue_idx_load(si, 2 * i + si + 2)

            lax.fori_loop(0, G, scores[si], 0)
            wait_v_gather(si)
            lax.fori_loop(0, G, scales[si], 0)
            issue_scatters(si)

            @pl.when(i < PAIRS - 1)
            def _():
                wait_idx_load(si)
                issue_gathers(si)

        for si in (0, 1):
            s = sets[si]
            base = ebase + si * C
            pltpu.sync_copy(i0_hbm.at[pl.ds(base, C)], s[0])
            pltpu.sync_copy(i1_hbm.at[pl.ds(base, C)], s[1])
            issue_gathers(si)

        def pair_body(i, carry):
            process(0, i)
            process(1, i)
            return carry

        lax.fori_loop(0, PAIRS, pair_body, 0)

        wait_scatters(0)
        wait_scatters(1)

        q_aT, k_aT, e0_aT, e1_aT, w_aT, vals_aT = seta[4:10]
        base = ebase + FCH * C
        pltpu.sync_copy(i0_hbm.at[pl.ds(base, TAIL_E)], i0_t)
        pltpu.sync_copy(i1_hbm.at[pl.ds(base, TAIL_E)], i1_t)
        pltpu.async_copy(q_hbm.at[i0_t], q_aT.at[pl.ds(0, TAIL_E)], gsem_a).wait()
        pltpu.async_copy(k_hbm.at[i1_t], k_aT.at[pl.ds(0, TAIL_E)], gsem_a).wait()
        pltpu.async_copy(v_hbm.at[i1_t], w_aT.at[pl.ds(0, TAIL_E)], gsem_a).wait()
        pltpu.async_copy(eigs_hbm.at[i0_t], e0_aT.at[pl.ds(0, TAIL_E)], gsem_a).wait()
        pltpu.async_copy(eigs_hbm.at[i1_t], e1_aT.at[pl.ds(0, TAIL_E)], gsem_a).wait()
        scores[0](0, 0)
        scales[0](0, 0)
        pltpu.async_copy(w_aT.at[pl.ds(0, TAIL_E)], acc_sh.at[i0_t], add=True, sem=ssem_a).wait()
        pltpu.async_copy(vals_aT.at[pl.ds(0, TAIL_E)], den_sh.at[i0_t], add=True, sem=ssem_a).wait()

        plsc.subcore_barrier()
        pltpu.sync_copy(
            acc_sh.at[pl.ds(sid * ROWS_PER_TILE, ROWS_PER_TILE)],
            acc_out.at[pl.ds(cid * N + sid * ROWS_PER_TILE, ROWS_PER_TILE)],
        )

        def den_out_copy(t, carry):
            r = sid * ROWS_PER_TILE + t * 16
            pltpu.sync_copy(den_sh.at[pl.ds(r, 16)], z16buf)
            pltpu.sync_copy(z16buf, den_out.at[pl.ds(cid * N + r, 16)])
            return carry

        lax.fori_loop(0, ROWS_PER_TILE // 16, den_out_copy, 0)

        @pl.when(sid == NS - 1)
        def _():
            pltpu.sync_copy(
                acc_sh.at[pl.ds(TAIL_ROW, N - TAIL_ROW)],
                acc_out.at[pl.ds(cid * N + TAIL_ROW, N - TAIL_ROW)],
            )
            pltpu.sync_copy(den_sh.at[pl.ds(TAIL_ROW, 16)], z16buf)
            pltpu.sync_copy(z16buf, den_out.at[pl.ds(cid * N + TAIL_ROW, 16)])

    return body(i0, i1, q, k, v, eigs, lam)


def _tc_combine(acc, den2):
    blk = 2000
    nb = N // blk

    def body(a0, a1, dn, o):
        d = jnp.sum(dn[...], axis=1)
        d = jnp.where(d == 0.0, 1.0, d)
        o[...] = (a0[...] + a1[...]) / d[:, None]

    return pl.pallas_call(
        body,
        grid=(nb,),
        in_specs=[
            pl.BlockSpec((blk, D), lambda i: (i, 0)),
            pl.BlockSpec((blk, D), lambda i: (i + nb, 0)),
            pl.BlockSpec((blk, NC), lambda i: (i, 0)),
        ],
        out_specs=pl.BlockSpec((blk, D), lambda i: (i, 0)),
        out_shape=jax.ShapeDtypeStruct((N, D), jnp.float32),
    )(acc, acc, den2)


def kernel(q, k, v, indices, eigs, lambda0):
    i0 = indices[0].astype(jnp.int32)
    i1 = indices[1].astype(jnp.int32)
    lam = jnp.full((16,), lambda0[0], jnp.float32)
    eigs_p = jnp.pad(eigs, ((0, 0), (0, D - ED)))
    acc, den = _sc_edge_pass(i0, i1, q, k, v, eigs_p, lam)
    return _tc_combine(acc, den.reshape(NC, N).T)
